# Initial kernel scaffold; baseline (speedup 1.0000x reference)
import jax
import jax.numpy as jnp
from jax.experimental import pallas as pl


def kernel(params, node_id, trianglelogic, squarelogic, triangle, notriangle, square, nosquare, triangle_neighbor, triangle_mask, square_neighbor, square_mask):
    raise NotImplementedError("write your pallas kernel here")


# speedup vs baseline: 1.8144x; 1.8144x over previous
import functools

import jax
import jax.numpy as jnp
from jax import lax
from jax.experimental import pallas as pl
from jax.experimental.pallas import tpu as pltpu
from jax.experimental.pallas import tpu_sc as plsc

D = 128
B = 256
T = 32
NLAYERS = 2
ROWS_PER_TENSOR = B * T * 3
NTENSORS = 3 * NLAYERS
NROWS = NTENSORS * ROWS_PER_TENSOR

NW = 32
ROWS_PER_WORKER = NROWS // NW
CHUNK = 128
CHUNKS_PER_WORKER = ROWS_PER_WORKER // CHUNK
NODE_PER_WORKER = B // NW

RB = 3072
BATCHES_PER_BLOCK = RB // (T * 3)
NBLOCKS = NROWS // RB


def _sc_gather_body(table, idx, nidx, out, nout, idx_v, nidx_v, rows_v, nrows_v, sem):
    wid = lax.axis_index("s") * 2 + lax.axis_index("c")
    pltpu.sync_copy(idx.at[wid], idx_v)

    def chunk(c, _):
        pltpu.async_copy(table.at[idx_v.at[c]], rows_v, sem).wait()
        pltpu.sync_copy(rows_v, out.at[wid, c])
        return 0

    lax.fori_loop(0, CHUNKS_PER_WORKER, chunk, 0)

    pltpu.sync_copy(nidx.at[wid], nidx_v)
    pltpu.async_copy(table.at[nidx_v], nrows_v, sem).wait()
    pltpu.sync_copy(nrows_v, nout.at[wid])


@functools.cache
def _get_sc_gather():
    return pl.kernel(
        _sc_gather_body,
        out_type=(
            jax.ShapeDtypeStruct((NW, CHUNKS_PER_WORKER, CHUNK, D), jnp.float32),
            jax.ShapeDtypeStruct((NW, NODE_PER_WORKER, D), jnp.float32),
        ),
        mesh=plsc.VectorSubcoreMesh(core_axis_name="c", subcore_axis_name="s"),
        scratch_types=[
            pltpu.VMEM((CHUNKS_PER_WORKER, CHUNK), jnp.int32),
            pltpu.VMEM((NODE_PER_WORKER,), jnp.int32),
            pltpu.VMEM((CHUNK, D), jnp.float32),
            pltpu.VMEM((NODE_PER_WORKER, D), jnp.float32),
            pltpu.SemaphoreType.DMA,
        ],
    )


def _mp_body(x_ref, ws_ref, wm_ref, b_ref, w_ref, m_ref):
    x = x_ref[...]
    ws = ws_ref[0]
    wm = wm_ref[0]
    b = b_ref[0]
    w = w_ref[...]
    s = x.reshape(RB // 3, 3, D).sum(axis=1)
    t = lax.dot(s, wm, preferred_element_type=jnp.float32) + b
    tb = jnp.broadcast_to(t[:, None, :], (RB // 3, 3, D)).reshape(RB, D)
    z = lax.dot(x, ws - wm, preferred_element_type=jnp.float32) + tb
    y = jnp.maximum(z, 0.0) * w
    num = y.reshape(BATCHES_PER_BLOCK, T * 3, D).sum(axis=1)
    den = w.reshape(BATCHES_PER_BLOCK, T * 3, 1).sum(axis=1)
    m_ref[...] = (num / (den + 1e-6))[None]


def _message_pass_means(rows, ws_all, wm_all, b_all, w_all):
    return pl.pallas_call(
        _mp_body,
        grid=(NBLOCKS,),
        in_specs=[
            pl.BlockSpec((RB, D), lambda i: (i, 0)),
            pl.BlockSpec((1, D, D), lambda i: (i // 8, 0, 0)),
            pl.BlockSpec((1, D, D), lambda i: (i // 8, 0, 0)),
            pl.BlockSpec((1, 1, D), lambda i: (i // 8, 0, 0)),
            pl.BlockSpec((RB, 1), lambda i: (i, 0)),
        ],
        out_specs=pl.BlockSpec((1, BATCHES_PER_BLOCK, D), lambda i: (i, 0, 0)),
        out_shape=jax.ShapeDtypeStruct((NBLOCKS, BATCHES_PER_BLOCK, D), jnp.float32),
    )(rows, ws_all, wm_all, b_all, w_all)


def _combine_body(m_ref, ne_ref, wg_ref, bg_ref, w1_ref, b1_ref, w2_ref, b2_ref, o_ref):
    aggs = []
    for k in range(NTENSORS):
        aggs.append(
            lax.dot(m_ref[k], wg_ref[k], preferred_element_type=jnp.float32)
            + bg_ref[k])
    t_mean = 0.25 * (aggs[0] - aggs[1] + aggs[3] - aggs[4])
    n_mean = 0.5 * (aggs[2] + aggs[5])
    h = (lax.dot(ne_ref[...], w1_ref[0], preferred_element_type=jnp.float32)
         + lax.dot(t_mean, w1_ref[1], preferred_element_type=jnp.float32)
         + lax.dot(n_mean, w1_ref[2], preferred_element_type=jnp.float32)
         + b1_ref[...])
    h1 = jnp.maximum(h, 0.0)
    o_ref[...] = jnp.sum(h1 * w2_ref[...], axis=1, keepdims=True) + b2_ref[...]


def _combine(m6, node_e, wg_all, bg_all, w1, b1, w2, b2):
    return pl.pallas_call(
        _combine_body,
        out_shape=jax.ShapeDtypeStruct((B, 1), jnp.float32),
    )(m6, node_e, wg_all, bg_all, w1, b1, w2, b2)


def kernel(params, node_id, trianglelogic, squarelogic, triangle, notriangle,
           square, nosquare, triangle_neighbor, triangle_mask, square_neighbor,
           square_mask):
    emb = params['embedding']

    idx_parts, w_parts, ws_l, wm_l, b_l, wg_l, bg_l = [], [], [], [], [], [], []
    ones = jnp.ones((ROWS_PER_TENSOR,), jnp.float32)
    for l in range(NLAYERS):
        idx_parts += [
            triangle[:, l].reshape(-1),
            notriangle[:, l].reshape(-1),
            triangle_neighbor[:, l].reshape(-1),
        ]
        w_parts += [ones, ones, triangle_mask[:, l].reshape(-1)]
        (ws_t, wm_t, bias_t) = params['tmp'][l][0]
        (ws_n, wm_n, bias_n) = params['tnp'][l][0]
        ws_l += [ws_t, ws_t, ws_n]
        wm_l += [wm_t, wm_t, wm_n]
        b_l += [bias_t, bias_t, bias_n]
        for name in ('tpa', 'tga', 'tna'):
            wg, bg = params[name][l][0]
            wg_l.append(wg)
            bg_l.append(bg)

    idx = jnp.concatenate(idx_parts).astype(jnp.int32).reshape(
        NW, CHUNKS_PER_WORKER, CHUNK)
    nidx = node_id.astype(jnp.int32).reshape(NW, NODE_PER_WORKER)
    w_all = jnp.concatenate(w_parts).reshape(NROWS, 1)

    ws_all = jnp.stack(ws_l)
    wm_all = jnp.stack(wm_l)
    b_all = jnp.stack(b_l).reshape(NTENSORS, 1, D)
    wg_all = jnp.stack(wg_l)
    bg_all = jnp.stack(bg_l).reshape(NTENSORS, 1, D)
    (w1, b1), (w2, b2) = params['combine']
    w1r = w1.reshape(3, D, D)
    b1r = b1.reshape(1, D)
    w2r = w2.reshape(1, D)
    b2r = b2.reshape(1, 1)

    rows, node_e = _get_sc_gather()(emb, idx, nidx)
    rows = rows.reshape(NROWS, D)
    node_e = node_e.reshape(B, D)
    m = _message_pass_means(rows, ws_all, wm_all, b_all, w_all)
    m6 = m.reshape(NTENSORS, B, D)
    return _combine(m6, node_e, wg_all, bg_all, w1r, b1r, w2r, b2r)


# speedup vs baseline: 1.8474x; 1.0182x over previous
import functools

import jax
import jax.numpy as jnp
from jax import lax
from jax.experimental import pallas as pl
from jax.experimental.pallas import tpu as pltpu
from jax.experimental.pallas import tpu_sc as plsc

D = 128
B = 256
T = 32
NLAYERS = 2
ROWS_PER_TENSOR = B * T * 3
NTENSORS = 3 * NLAYERS
NROWS = NTENSORS * ROWS_PER_TENSOR

NW = 32
ROWS_PER_WORKER = NROWS // NW
CHUNK = 128
CHUNKS_PER_WORKER = ROWS_PER_WORKER // CHUNK
NODE_PER_WORKER = B // NW

RB = 3072
BATCHES_PER_BLOCK = RB // (T * 3)
NBLOCKS = NROWS // RB


def _sc_gather_body(table, idx, nidx, out, nout, idx_v, nidx_v, rows_a, rows_b,
                    nrows_v, sem_a, sem_b):
    wid = lax.axis_index("s") * 2 + lax.axis_index("c")
    pltpu.sync_copy(idx.at[wid], idx_v)

    pltpu.async_copy(table.at[idx_v.at[0]], rows_a, sem_a)

    def pair(i, _):
        c = 2 * i
        pltpu.async_copy(table.at[idx_v.at[c + 1]], rows_b, sem_b)
        pltpu.make_async_copy(table.at[idx_v.at[c]], rows_a, sem_a).wait()
        pltpu.sync_copy(rows_a, out.at[wid, c])

        @pl.when(c + 2 < CHUNKS_PER_WORKER)
        def _():
            pltpu.async_copy(table.at[idx_v.at[c + 2]], rows_a, sem_a)

        pltpu.make_async_copy(table.at[idx_v.at[c + 1]], rows_b, sem_b).wait()
        pltpu.sync_copy(rows_b, out.at[wid, c + 1])
        return 0

    lax.fori_loop(0, CHUNKS_PER_WORKER // 2, pair, 0)

    pltpu.sync_copy(nidx.at[wid], nidx_v)
    pltpu.async_copy(table.at[nidx_v], nrows_v, sem_a).wait()
    pltpu.sync_copy(nrows_v, nout.at[wid])


@functools.cache
def _get_sc_gather():
    return pl.kernel(
        _sc_gather_body,
        out_type=(
            jax.ShapeDtypeStruct((NW, CHUNKS_PER_WORKER, CHUNK, D), jnp.float32),
            jax.ShapeDtypeStruct((NW, NODE_PER_WORKER, D), jnp.float32),
        ),
        mesh=plsc.VectorSubcoreMesh(core_axis_name="c", subcore_axis_name="s"),
        scratch_types=[
            pltpu.VMEM((CHUNKS_PER_WORKER, CHUNK), jnp.int32),
            pltpu.VMEM((NODE_PER_WORKER,), jnp.int32),
            pltpu.VMEM((CHUNK, D), jnp.float32),
            pltpu.VMEM((CHUNK, D), jnp.float32),
            pltpu.VMEM((NODE_PER_WORKER, D), jnp.float32),
            pltpu.SemaphoreType.DMA,
            pltpu.SemaphoreType.DMA,
        ],
    )


def _mp_body(x_ref, ws_ref, wm_ref, b_ref, w_ref, m_ref):
    x = x_ref[...]
    ws = ws_ref[0]
    wm = wm_ref[0]
    b = b_ref[0]
    w = w_ref[...]
    s = x.reshape(RB // 3, 3, D).sum(axis=1)
    t = lax.dot(s, wm, preferred_element_type=jnp.float32) + b
    tb = jnp.broadcast_to(t[:, None, :], (RB // 3, 3, D)).reshape(RB, D)
    z = lax.dot(x, ws - wm, preferred_element_type=jnp.float32) + tb
    y = jnp.maximum(z, 0.0) * w
    num = y.reshape(BATCHES_PER_BLOCK, T * 3, D).sum(axis=1)
    den = w.reshape(BATCHES_PER_BLOCK, T * 3, 1).sum(axis=1)
    m_ref[...] = (num / (den + 1e-6))[None]


def _message_pass_means(rows, ws_all, wm_all, b_all, w_all):
    return pl.pallas_call(
        _mp_body,
        grid=(NBLOCKS,),
        in_specs=[
            pl.BlockSpec((RB, D), lambda i: (i, 0)),
            pl.BlockSpec((1, D, D), lambda i: (i // 8, 0, 0)),
            pl.BlockSpec((1, D, D), lambda i: (i // 8, 0, 0)),
            pl.BlockSpec((1, 1, D), lambda i: (i // 8, 0, 0)),
            pl.BlockSpec((RB, 1), lambda i: (i, 0)),
        ],
        out_specs=pl.BlockSpec((1, BATCHES_PER_BLOCK, D), lambda i: (i, 0, 0)),
        out_shape=jax.ShapeDtypeStruct((NBLOCKS, BATCHES_PER_BLOCK, D), jnp.float32),
    )(rows, ws_all, wm_all, b_all, w_all)


def _combine_body(m_ref, ne_ref, wg_ref, bg_ref, w1_ref, b1_ref, w2_ref, b2_ref, o_ref):
    aggs = []
    for k in range(NTENSORS):
        aggs.append(
            lax.dot(m_ref[k], wg_ref[k], preferred_element_type=jnp.float32)
            + bg_ref[k])
    t_mean = 0.25 * (aggs[0] - aggs[1] + aggs[3] - aggs[4])
    n_mean = 0.5 * (aggs[2] + aggs[5])
    h = (lax.dot(ne_ref[...], w1_ref[0], preferred_element_type=jnp.float32)
         + lax.dot(t_mean, w1_ref[1], preferred_element_type=jnp.float32)
         + lax.dot(n_mean, w1_ref[2], preferred_element_type=jnp.float32)
         + b1_ref[...])
    h1 = jnp.maximum(h, 0.0)
    o_ref[...] = jnp.sum(h1 * w2_ref[...], axis=1, keepdims=True) + b2_ref[...]


def _combine(m6, node_e, wg_all, bg_all, w1, b1, w2, b2):
    return pl.pallas_call(
        _combine_body,
        out_shape=jax.ShapeDtypeStruct((B, 1), jnp.float32),
    )(m6, node_e, wg_all, bg_all, w1, b1, w2, b2)


def kernel(params, node_id, trianglelogic, squarelogic, triangle, notriangle,
           square, nosquare, triangle_neighbor, triangle_mask, square_neighbor,
           square_mask):
    emb = params['embedding']

    idx_parts, w_parts, ws_l, wm_l, b_l, wg_l, bg_l = [], [], [], [], [], [], []
    ones = jnp.ones((ROWS_PER_TENSOR,), jnp.float32)
    for l in range(NLAYERS):
        idx_parts += [
            triangle[:, l].reshape(-1),
            notriangle[:, l].reshape(-1),
            triangle_neighbor[:, l].reshape(-1),
        ]
        w_parts += [ones, ones, triangle_mask[:, l].reshape(-1)]
        (ws_t, wm_t, bias_t) = params['tmp'][l][0]
        (ws_n, wm_n, bias_n) = params['tnp'][l][0]
        ws_l += [ws_t, ws_t, ws_n]
        wm_l += [wm_t, wm_t, wm_n]
        b_l += [bias_t, bias_t, bias_n]
        for name in ('tpa', 'tga', 'tna'):
            wg, bg = params[name][l][0]
            wg_l.append(wg)
            bg_l.append(bg)

    idx = jnp.concatenate(idx_parts).astype(jnp.int32).reshape(
        NW, CHUNKS_PER_WORKER, CHUNK)
    nidx = node_id.astype(jnp.int32).reshape(NW, NODE_PER_WORKER)
    w_all = jnp.concatenate(w_parts).reshape(NROWS, 1)

    ws_all = jnp.stack(ws_l)
    wm_all = jnp.stack(wm_l)
    b_all = jnp.stack(b_l).reshape(NTENSORS, 1, D)
    wg_all = jnp.stack(wg_l)
    bg_all = jnp.stack(bg_l).reshape(NTENSORS, 1, D)
    (w1, b1), (w2, b2) = params['combine']
    w1r = w1.reshape(3, D, D)
    b1r = b1.reshape(1, D)
    w2r = w2.reshape(1, D)
    b2r = b2.reshape(1, 1)

    rows, node_e = _get_sc_gather()(emb, idx, nidx)
    rows = rows.reshape(NROWS, D)
    node_e = node_e.reshape(B, D)
    m = _message_pass_means(rows, ws_all, wm_all, b_all, w_all)
    m6 = m.reshape(NTENSORS, B, D)
    return _combine(m6, node_e, wg_all, bg_all, w1r, b1r, w2r, b2r)


# speedup vs baseline: 1.9085x; 1.0331x over previous
import functools

import jax
import jax.numpy as jnp
from jax import lax
from jax.experimental import pallas as pl
from jax.experimental.pallas import tpu as pltpu
from jax.experimental.pallas import tpu_sc as plsc

D = 128
B = 256
T = 32
NLAYERS = 2
ROWS_PER_TENSOR = B * T * 3
TENSORS_PER_HALF = 3
HROWS = TENSORS_PER_HALF * ROWS_PER_TENSOR

NW = 32
CHUNK = 128
H_CHUNKS = HROWS // (NW * CHUNK)
NODE_PER_WORKER = B // NW

RB = 3072
BATCHES_PER_BLOCK = RB // (T * 3)
H_BLOCKS = HROWS // RB


def _sc_gather_half_body(with_node, table, idx, nidx, out, nout, idx_v, nidx_v,
                         rows_a, rows_b, nrows_v, sem_a, sem_b):
    wid = lax.axis_index("s") * 2 + lax.axis_index("c")
    pltpu.sync_copy(idx.at[wid], idx_v)

    pltpu.async_copy(table.at[idx_v.at[0]], rows_a, sem_a)

    def pair(i, _):
        c = 2 * i
        pltpu.async_copy(table.at[idx_v.at[c + 1]], rows_b, sem_b)
        pltpu.make_async_copy(table.at[idx_v.at[c]], rows_a, sem_a).wait()
        pltpu.sync_copy(rows_a, out.at[wid, c])

        @pl.when(c + 2 < H_CHUNKS)
        def _():
            pltpu.async_copy(table.at[idx_v.at[c + 2]], rows_a, sem_a)

        pltpu.make_async_copy(table.at[idx_v.at[c + 1]], rows_b, sem_b).wait()
        pltpu.sync_copy(rows_b, out.at[wid, c + 1])
        return 0

    lax.fori_loop(0, H_CHUNKS // 2, pair, 0)

    if with_node:
        pltpu.sync_copy(nidx.at[wid], nidx_v)
        pltpu.async_copy(table.at[nidx_v], nrows_v, sem_a).wait()
        pltpu.sync_copy(nrows_v, nout.at[wid])


@functools.cache
def _get_sc_gather(with_node):
    out_type = [jax.ShapeDtypeStruct((NW, H_CHUNKS, CHUNK, D), jnp.float32)]
    scratch = [
        pltpu.VMEM((H_CHUNKS, CHUNK), jnp.int32),
        pltpu.VMEM((NODE_PER_WORKER,), jnp.int32),
        pltpu.VMEM((CHUNK, D), jnp.float32),
        pltpu.VMEM((CHUNK, D), jnp.float32),
        pltpu.VMEM((NODE_PER_WORKER, D), jnp.float32),
        pltpu.SemaphoreType.DMA,
        pltpu.SemaphoreType.DMA,
    ]
    if with_node:
        out_type.append(jax.ShapeDtypeStruct((NW, NODE_PER_WORKER, D), jnp.float32))

        def body(table, idx, nidx, out, nout, idx_v, nidx_v, ra, rb, nv, sa, sb):
            _sc_gather_half_body(True, table, idx, nidx, out, nout, idx_v,
                                 nidx_v, ra, rb, nv, sa, sb)
    else:
        def body(table, idx, out, idx_v, nidx_v, ra, rb, nv, sa, sb):
            _sc_gather_half_body(False, table, idx, None, out, None, idx_v,
                                 nidx_v, ra, rb, nv, sa, sb)

    return pl.kernel(
        body,
        out_type=tuple(out_type),
        mesh=plsc.VectorSubcoreMesh(core_axis_name="c", subcore_axis_name="s"),
        scratch_types=scratch,
    )


def _mp_body(x_ref, ws_ref, wm_ref, b_ref, w_ref, m_ref):
    x = x_ref[...]
    ws = ws_ref[0]
    wm = wm_ref[0]
    b = b_ref[0]
    w = w_ref[...]
    s = x.reshape(RB // 3, 3, D).sum(axis=1)
    t = lax.dot(s, wm, preferred_element_type=jnp.float32) + b
    tb = jnp.broadcast_to(t[:, None, :], (RB // 3, 3, D)).reshape(RB, D)
    z = lax.dot(x, ws - wm, preferred_element_type=jnp.float32) + tb
    y = jnp.maximum(z, 0.0) * w
    num = y.reshape(BATCHES_PER_BLOCK, T * 3, D).sum(axis=1)
    den = w.reshape(BATCHES_PER_BLOCK, T * 3, 1).sum(axis=1)
    m_ref[...] = (num / (den + 1e-6))[None]


def _message_pass_means(rows, ws_all, wm_all, b_all, w_all):
    return pl.pallas_call(
        _mp_body,
        grid=(H_BLOCKS,),
        in_specs=[
            pl.BlockSpec((RB, D), lambda i: (i, 0)),
            pl.BlockSpec((1, D, D), lambda i: (i // 8, 0, 0)),
            pl.BlockSpec((1, D, D), lambda i: (i // 8, 0, 0)),
            pl.BlockSpec((1, 1, D), lambda i: (i // 8, 0, 0)),
            pl.BlockSpec((RB, 1), lambda i: (i, 0)),
        ],
        out_specs=pl.BlockSpec((1, BATCHES_PER_BLOCK, D), lambda i: (i, 0, 0)),
        out_shape=jax.ShapeDtypeStruct((H_BLOCKS, BATCHES_PER_BLOCK, D), jnp.float32),
    )(rows, ws_all, wm_all, b_all, w_all)


def _combine_body(ma_ref, mb_ref, ne_ref, wg_ref, bg_ref, w1_ref, b1_ref,
                  w2_ref, b2_ref, o_ref):
    def agg(m_ref, half, k):
        return (lax.dot(m_ref[k], wg_ref[half * 3 + k],
                        preferred_element_type=jnp.float32)
                + bg_ref[half * 3 + k])

    t_mean = 0.25 * (agg(ma_ref, 0, 0) - agg(ma_ref, 0, 1)
                     + agg(mb_ref, 1, 0) - agg(mb_ref, 1, 1))
    n_mean = 0.5 * (agg(ma_ref, 0, 2) + agg(mb_ref, 1, 2))
    h = (lax.dot(ne_ref[...], w1_ref[0], preferred_element_type=jnp.float32)
         + lax.dot(t_mean, w1_ref[1], preferred_element_type=jnp.float32)
         + lax.dot(n_mean, w1_ref[2], preferred_element_type=jnp.float32)
         + b1_ref[...])
    h1 = jnp.maximum(h, 0.0)
    o_ref[...] = jnp.sum(h1 * w2_ref[...], axis=1, keepdims=True) + b2_ref[...]


def _combine(ma, mb, node_e, wg_all, bg_all, w1, b1, w2, b2):
    return pl.pallas_call(
        _combine_body,
        out_shape=jax.ShapeDtypeStruct((B, 1), jnp.float32),
    )(ma, mb, node_e, wg_all, bg_all, w1, b1, w2, b2)


def kernel(params, node_id, trianglelogic, squarelogic, triangle, notriangle,
           square, nosquare, triangle_neighbor, triangle_mask, square_neighbor,
           square_mask):
    emb = params['embedding']

    idx_halves, w_halves, ws_h, wm_h, b_h = [], [], [], [], []
    wg_l, bg_l = [], []
    ones = jnp.ones((ROWS_PER_TENSOR,), jnp.float32)
    for l in range(NLAYERS):
        idx_halves.append(jnp.concatenate([
            triangle[:, l].reshape(-1),
            notriangle[:, l].reshape(-1),
            triangle_neighbor[:, l].reshape(-1),
        ]).astype(jnp.int32).reshape(NW, H_CHUNKS, CHUNK))
        w_halves.append(jnp.concatenate(
            [ones, ones, triangle_mask[:, l].reshape(-1)]).reshape(HROWS, 1))
        (ws_t, wm_t, bias_t) = params['tmp'][l][0]
        (ws_n, wm_n, bias_n) = params['tnp'][l][0]
        ws_h.append(jnp.stack([ws_t, ws_t, ws_n]))
        wm_h.append(jnp.stack([wm_t, wm_t, wm_n]))
        b_h.append(jnp.stack([bias_t, bias_t, bias_n]).reshape(3, 1, D))
        for name in ('tpa', 'tga', 'tna'):
            wg, bg = params[name][l][0]
            wg_l.append(wg)
            bg_l.append(bg)

    nidx = node_id.astype(jnp.int32).reshape(NW, NODE_PER_WORKER)
    wg_all = jnp.stack(wg_l)
    bg_all = jnp.stack(bg_l).reshape(2 * TENSORS_PER_HALF, 1, D)
    (w1, b1), (w2, b2) = params['combine']
    w1r = w1.reshape(3, D, D)
    b1r = b1.reshape(1, D)
    w2r = w2.reshape(1, D)
    b2r = b2.reshape(1, 1)

    rows0, node_e = _get_sc_gather(True)(emb, idx_halves[0], nidx)
    rows1, = _get_sc_gather(False)(emb, idx_halves[1])
    node_e = node_e.reshape(B, D)
    m = []
    for l, rows in enumerate((rows0, rows1)):
        m.append(_message_pass_means(
            rows.reshape(HROWS, D), ws_h[l], wm_h[l], b_h[l], w_halves[l]
        ).reshape(TENSORS_PER_HALF, B, D))
    return _combine(m[0], m[1], node_e, wg_all, bg_all, w1r, b1r, w2r, b2r)


# speedup vs baseline: 3.4956x; 1.8316x over previous
import functools

import jax
import jax.numpy as jnp
from jax import lax
from jax.experimental import pallas as pl
from jax.experimental.pallas import tpu as pltpu
from jax.experimental.pallas import tpu_sc as plsc

D = 128
B = 256
T = 32
NLAYERS = 2
ROWS_PER_TENSOR = B * T * 3
TENSORS_PER_HALF = 3
HROWS = TENSORS_PER_HALF * ROWS_PER_TENSOR

NW = 32
CHUNK = 128
H_CHUNKS = HROWS // (NW * CHUNK)
NODE_PER_WORKER = B // NW

RB = 3072
BATCHES_PER_BLOCK = RB // (T * 3)
H_BLOCKS = HROWS // RB


def _sc_gather_half_body(with_node, table, idx, nidx, out, nout, idx_v, nidx_v,
                         rows_a, rows_b, nrows_v, sem_a, sem_b):
    wid = lax.axis_index("s") * 2 + lax.axis_index("c")
    pltpu.sync_copy(idx.at[wid], idx_v)

    pltpu.async_copy(table.at[idx_v.at[0]], rows_a, sem_a)

    def pair(i, _):
        c = 2 * i
        pltpu.async_copy(table.at[idx_v.at[c + 1]], rows_b, sem_b)
        pltpu.make_async_copy(table.at[idx_v.at[c]], rows_a, sem_a).wait()
        pltpu.sync_copy(rows_a, out.at[wid, c])

        @pl.when(c + 2 < H_CHUNKS)
        def _():
            pltpu.async_copy(table.at[idx_v.at[c + 2]], rows_a, sem_a)

        pltpu.make_async_copy(table.at[idx_v.at[c + 1]], rows_b, sem_b).wait()
        pltpu.sync_copy(rows_b, out.at[wid, c + 1])
        return 0

    lax.fori_loop(0, H_CHUNKS // 2, pair, 0)

    if with_node:
        pltpu.sync_copy(nidx.at[wid], nidx_v)
        pltpu.async_copy(table.at[nidx_v], nrows_v, sem_a).wait()
        pltpu.sync_copy(nrows_v, nout.at[wid])


@functools.cache
def _get_sc_gather(with_node):
    out_type = [jax.ShapeDtypeStruct((NW, H_CHUNKS, CHUNK, D), jnp.float32)]
    scratch = [
        pltpu.VMEM((H_CHUNKS, CHUNK), jnp.int32),
        pltpu.VMEM((NODE_PER_WORKER,), jnp.int32),
        pltpu.VMEM((CHUNK, D), jnp.float32),
        pltpu.VMEM((CHUNK, D), jnp.float32),
        pltpu.VMEM((NODE_PER_WORKER, D), jnp.float32),
        pltpu.SemaphoreType.DMA,
        pltpu.SemaphoreType.DMA,
    ]
    if with_node:
        out_type.append(jax.ShapeDtypeStruct((NW, NODE_PER_WORKER, D), jnp.float32))

        def body(table, idx, nidx, out, nout, idx_v, nidx_v, ra, rb, nv, sa, sb):
            _sc_gather_half_body(True, table, idx, nidx, out, nout, idx_v,
                                 nidx_v, ra, rb, nv, sa, sb)
    else:
        def body(table, idx, out, idx_v, nidx_v, ra, rb, nv, sa, sb):
            _sc_gather_half_body(False, table, idx, None, out, None, idx_v,
                                 nidx_v, ra, rb, nv, sa, sb)

    return pl.kernel(
        body,
        out_type=tuple(out_type),
        mesh=plsc.VectorSubcoreMesh(core_axis_name="c", subcore_axis_name="s"),
        scratch_types=scratch,
    )


def _mp_body(x_ref, ws_ref, wm_ref, b_ref, w_ref, m_ref):
    x3 = x_ref[0]
    ws = ws_ref[0]
    wm = wm_ref[0]
    b = b_ref[0]
    w3 = w_ref[0]
    g = RB // 3
    s = x3[0] + x3[1] + x3[2]
    t = lax.dot(s, wm, preferred_element_type=jnp.float32) + b
    xf = x3.reshape(RB, D)
    tb = jnp.broadcast_to(t[None], (3, g, D)).reshape(RB, D)
    z = lax.dot(xf, ws - wm, preferred_element_type=jnp.float32) + tb
    y = (jnp.maximum(z, 0.0) * w3.reshape(RB, 1)).reshape(3, g, D)
    part = y[0] + y[1] + y[2]
    num = part.reshape(BATCHES_PER_BLOCK, T, D).sum(axis=1)
    wsum = w3[0] + w3[1] + w3[2]
    den = wsum.reshape(BATCHES_PER_BLOCK, T, 1).sum(axis=1)
    m_ref[...] = (num / (den + 1e-6))[None]


def _message_pass_means(rows, ws_all, wm_all, b_all, w_all):
    return pl.pallas_call(
        _mp_body,
        grid=(H_BLOCKS,),
        in_specs=[
            pl.BlockSpec((1, 3, RB // 3, D), lambda i: (i // 8, 0, i % 8, 0)),
            pl.BlockSpec((1, D, D), lambda i: (i // 8, 0, 0)),
            pl.BlockSpec((1, D, D), lambda i: (i // 8, 0, 0)),
            pl.BlockSpec((1, 1, D), lambda i: (i // 8, 0, 0)),
            pl.BlockSpec((1, 3, RB // 3, 1), lambda i: (i // 8, 0, i % 8, 0)),
        ],
        out_specs=pl.BlockSpec((1, BATCHES_PER_BLOCK, D), lambda i: (i, 0, 0)),
        out_shape=jax.ShapeDtypeStruct((H_BLOCKS, BATCHES_PER_BLOCK, D), jnp.float32),
    )(rows, ws_all, wm_all, b_all, w_all)


def _combine_body(ma_ref, mb_ref, ne_ref, wg_ref, bg_ref, w1_ref, b1_ref,
                  w2_ref, b2_ref, o_ref):
    def agg(m_ref, half, k):
        return (lax.dot(m_ref[k], wg_ref[half * 3 + k],
                        preferred_element_type=jnp.float32)
                + bg_ref[half * 3 + k])

    t_mean = 0.25 * (agg(ma_ref, 0, 0) - agg(ma_ref, 0, 1)
                     + agg(mb_ref, 1, 0) - agg(mb_ref, 1, 1))
    n_mean = 0.5 * (agg(ma_ref, 0, 2) + agg(mb_ref, 1, 2))
    h = (lax.dot(ne_ref[...], w1_ref[0], preferred_element_type=jnp.float32)
         + lax.dot(t_mean, w1_ref[1], preferred_element_type=jnp.float32)
         + lax.dot(n_mean, w1_ref[2], preferred_element_type=jnp.float32)
         + b1_ref[...])
    h1 = jnp.maximum(h, 0.0)
    o_ref[...] = jnp.sum(h1 * w2_ref[...], axis=1, keepdims=True) + b2_ref[...]


def _combine(ma, mb, node_e, wg_all, bg_all, w1, b1, w2, b2):
    return pl.pallas_call(
        _combine_body,
        out_shape=jax.ShapeDtypeStruct((B, 1), jnp.float32),
    )(ma, mb, node_e, wg_all, bg_all, w1, b1, w2, b2)


def kernel(params, node_id, trianglelogic, squarelogic, triangle, notriangle,
           square, nosquare, triangle_neighbor, triangle_mask, square_neighbor,
           square_mask):
    emb = params['embedding']

    idx_halves, w_halves, ws_h, wm_h, b_h = [], [], [], [], []
    wg_l, bg_l = [], []
    ones = jnp.ones((ROWS_PER_TENSOR,), jnp.float32)

    def node_major(a):
        return jnp.transpose(a, (2, 0, 1)).reshape(-1)

    for l in range(NLAYERS):
        idx_halves.append(jnp.concatenate([
            node_major(triangle[:, l]),
            node_major(notriangle[:, l]),
            node_major(triangle_neighbor[:, l]),
        ]).astype(jnp.int32).reshape(NW, H_CHUNKS, CHUNK))
        w_halves.append(jnp.concatenate(
            [ones, ones, node_major(triangle_mask[:, l])]
        ).reshape(TENSORS_PER_HALF, 3, B * T, 1))
        (ws_t, wm_t, bias_t) = params['tmp'][l][0]
        (ws_n, wm_n, bias_n) = params['tnp'][l][0]
        ws_h.append(jnp.stack([ws_t, ws_t, ws_n]))
        wm_h.append(jnp.stack([wm_t, wm_t, wm_n]))
        b_h.append(jnp.stack([bias_t, bias_t, bias_n]).reshape(3, 1, D))
        for name in ('tpa', 'tga', 'tna'):
            wg, bg = params[name][l][0]
            wg_l.append(wg)
            bg_l.append(bg)

    nidx = node_id.astype(jnp.int32).reshape(NW, NODE_PER_WORKER)
    wg_all = jnp.stack(wg_l)
    bg_all = jnp.stack(bg_l).reshape(2 * TENSORS_PER_HALF, 1, D)
    (w1, b1), (w2, b2) = params['combine']
    w1r = w1.reshape(3, D, D)
    b1r = b1.reshape(1, D)
    w2r = w2.reshape(1, D)
    b2r = b2.reshape(1, 1)

    rows0, node_e = _get_sc_gather(True)(emb, idx_halves[0], nidx)
    rows1, = _get_sc_gather(False)(emb, idx_halves[1])
    node_e = node_e.reshape(B, D)
    m = []
    for l, rows in enumerate((rows0, rows1)):
        m.append(_message_pass_means(
            rows.reshape(TENSORS_PER_HALF, 3, B * T, D),
            ws_h[l], wm_h[l], b_h[l], w_halves[l]
        ).reshape(TENSORS_PER_HALF, B, D))
    return _combine(m[0], m[1], node_e, wg_all, bg_all, w1r, b1r, w2r, b2r)


# speedup vs baseline: 3.4986x; 1.0009x over previous
import functools

import jax
import jax.numpy as jnp
from jax import lax
from jax.experimental import pallas as pl
from jax.experimental.pallas import tpu as pltpu
from jax.experimental.pallas import tpu_sc as plsc

D = 128
B = 256
T = 32
NLAYERS = 2
ROWS_PER_TENSOR = B * T * 3
TENSORS_PER_HALF = 3
HROWS = TENSORS_PER_HALF * ROWS_PER_TENSOR

NW = 32
CHUNK = 128
H_CHUNKS = HROWS // (NW * CHUNK)
NODE_PER_WORKER = B // NW

RB = 3072
BATCHES_PER_BLOCK = RB // (T * 3)
H_BLOCKS = HROWS // RB


def _sc_gather_half_body(with_node, table, idx, nidx, out, nout, idx_v, nidx_v,
                         rows_a, rows_b, nrows_v, sem_a, sem_b):
    wid = lax.axis_index("s") * 2 + lax.axis_index("c")
    pltpu.sync_copy(idx.at[wid], idx_v)

    def store(c, rows):
        ch = wid * H_CHUNKS + c
        pltpu.sync_copy(rows, out.at[ch // 192, (ch // 64) % 3, ch % 64])

    pltpu.async_copy(table.at[idx_v.at[0]], rows_a, sem_a)

    def pair(i, _):
        c = 2 * i
        pltpu.async_copy(table.at[idx_v.at[c + 1]], rows_b, sem_b)
        pltpu.make_async_copy(table.at[idx_v.at[c]], rows_a, sem_a).wait()
        store(c, rows_a)

        @pl.when(c + 2 < H_CHUNKS)
        def _():
            pltpu.async_copy(table.at[idx_v.at[c + 2]], rows_a, sem_a)

        pltpu.make_async_copy(table.at[idx_v.at[c + 1]], rows_b, sem_b).wait()
        store(c + 1, rows_b)
        return 0

    lax.fori_loop(0, H_CHUNKS // 2, pair, 0)

    if with_node:
        pltpu.sync_copy(nidx.at[wid], nidx_v)
        pltpu.async_copy(table.at[nidx_v], nrows_v, sem_a).wait()
        pltpu.sync_copy(nrows_v, nout.at[wid])


@functools.cache
def _get_sc_gather(with_node):
    out_type = [jax.ShapeDtypeStruct(
        (TENSORS_PER_HALF, 3, HROWS // (9 * CHUNK), CHUNK, D), jnp.float32)]
    scratch = [
        pltpu.VMEM((H_CHUNKS, CHUNK), jnp.int32),
        pltpu.VMEM((NODE_PER_WORKER,), jnp.int32),
        pltpu.VMEM((CHUNK, D), jnp.float32),
        pltpu.VMEM((CHUNK, D), jnp.float32),
        pltpu.VMEM((NODE_PER_WORKER, D), jnp.float32),
        pltpu.SemaphoreType.DMA,
        pltpu.SemaphoreType.DMA,
    ]
    if with_node:
        out_type.append(jax.ShapeDtypeStruct((NW, NODE_PER_WORKER, D), jnp.float32))

        def body(table, idx, nidx, out, nout, idx_v, nidx_v, ra, rb, nv, sa, sb):
            _sc_gather_half_body(True, table, idx, nidx, out, nout, idx_v,
                                 nidx_v, ra, rb, nv, sa, sb)
    else:
        def body(table, idx, out, idx_v, nidx_v, ra, rb, nv, sa, sb):
            _sc_gather_half_body(False, table, idx, None, out, None, idx_v,
                                 nidx_v, ra, rb, nv, sa, sb)

    return pl.kernel(
        body,
        out_type=tuple(out_type),
        mesh=plsc.VectorSubcoreMesh(core_axis_name="c", subcore_axis_name="s"),
        scratch_types=scratch,
    )


def _mp_body(x_ref, ws_ref, wm_ref, b_ref, w_ref, m_ref):
    g = RB // 3
    x3 = x_ref[0].reshape(3, g, D)
    ws = ws_ref[0]
    wm = wm_ref[0]
    b = b_ref[0]
    w3 = w_ref[0]
    s = x3[0] + x3[1] + x3[2]
    t = lax.dot(s, wm, preferred_element_type=jnp.float32) + b
    xf = x3.reshape(RB, D)
    tb = jnp.broadcast_to(t[None], (3, g, D)).reshape(RB, D)
    z = lax.dot(xf, ws - wm, preferred_element_type=jnp.float32) + tb
    y = (jnp.maximum(z, 0.0) * w3.reshape(RB, 1)).reshape(3, g, D)
    part = y[0] + y[1] + y[2]
    num = part.reshape(BATCHES_PER_BLOCK, T, D).sum(axis=1)
    wsum = w3[0] + w3[1] + w3[2]
    den = wsum.reshape(BATCHES_PER_BLOCK, T, 1).sum(axis=1)
    m_ref[...] = (num / (den + 1e-6))[None]


def _message_pass_means(rows, ws_all, wm_all, b_all, w_all):
    blks = RB // 3 // CHUNK
    return pl.pallas_call(
        _mp_body,
        grid=(H_BLOCKS,),
        in_specs=[
            pl.BlockSpec((1, 3, blks, CHUNK, D), lambda i: (i // 8, 0, i % 8, 0, 0)),
            pl.BlockSpec((1, D, D), lambda i: (i // 8, 0, 0)),
            pl.BlockSpec((1, D, D), lambda i: (i // 8, 0, 0)),
            pl.BlockSpec((1, 1, D), lambda i: (i // 8, 0, 0)),
            pl.BlockSpec((1, 3, RB // 3, 1), lambda i: (i // 8, 0, i % 8, 0)),
        ],
        out_specs=pl.BlockSpec((1, BATCHES_PER_BLOCK, D), lambda i: (i, 0, 0)),
        out_shape=jax.ShapeDtypeStruct((H_BLOCKS, BATCHES_PER_BLOCK, D), jnp.float32),
    )(rows, ws_all, wm_all, b_all, w_all)


def _combine_body(ma_ref, mb_ref, ne_ref, wg_ref, bg_ref, w1_ref, b1_ref,
                  w2_ref, b2_ref, o_ref):
    def agg(m_ref, half, k):
        return (lax.dot(m_ref[k], wg_ref[half * 3 + k],
                        preferred_element_type=jnp.float32)
                + bg_ref[half * 3 + k])

    t_mean = 0.25 * (agg(ma_ref, 0, 0) - agg(ma_ref, 0, 1)
                     + agg(mb_ref, 1, 0) - agg(mb_ref, 1, 1))
    n_mean = 0.5 * (agg(ma_ref, 0, 2) + agg(mb_ref, 1, 2))
    h = (lax.dot(ne_ref[...], w1_ref[0], preferred_element_type=jnp.float32)
         + lax.dot(t_mean, w1_ref[1], preferred_element_type=jnp.float32)
         + lax.dot(n_mean, w1_ref[2], preferred_element_type=jnp.float32)
         + b1_ref[...])
    h1 = jnp.maximum(h, 0.0)
    o_ref[...] = jnp.sum(h1 * w2_ref[...], axis=1, keepdims=True) + b2_ref[...]


def _combine(ma, mb, node_e, wg_all, bg_all, w1, b1, w2, b2):
    return pl.pallas_call(
        _combine_body,
        out_shape=jax.ShapeDtypeStruct((B, 1), jnp.float32),
    )(ma, mb, node_e, wg_all, bg_all, w1, b1, w2, b2)


def kernel(params, node_id, trianglelogic, squarelogic, triangle, notriangle,
           square, nosquare, triangle_neighbor, triangle_mask, square_neighbor,
           square_mask):
    emb = params['embedding']

    idx_halves, w_halves, ws_h, wm_h, b_h = [], [], [], [], []
    wg_l, bg_l = [], []
    ones = jnp.ones((ROWS_PER_TENSOR,), jnp.float32)

    def node_major(a):
        return jnp.transpose(a, (2, 0, 1)).reshape(-1)

    for l in range(NLAYERS):
        idx_halves.append(jnp.concatenate([
            node_major(triangle[:, l]),
            node_major(notriangle[:, l]),
            node_major(triangle_neighbor[:, l]),
        ]).astype(jnp.int32).reshape(NW, H_CHUNKS, CHUNK))
        w_halves.append(jnp.concatenate(
            [ones, ones, node_major(triangle_mask[:, l])]
        ).reshape(TENSORS_PER_HALF, 3, B * T, 1))
        (ws_t, wm_t, bias_t) = params['tmp'][l][0]
        (ws_n, wm_n, bias_n) = params['tnp'][l][0]
        ws_h.append(jnp.stack([ws_t, ws_t, ws_n]))
        wm_h.append(jnp.stack([wm_t, wm_t, wm_n]))
        b_h.append(jnp.stack([bias_t, bias_t, bias_n]).reshape(3, 1, D))
        for name in ('tpa', 'tga', 'tna'):
            wg, bg = params[name][l][0]
            wg_l.append(wg)
            bg_l.append(bg)

    nidx = node_id.astype(jnp.int32).reshape(NW, NODE_PER_WORKER)
    wg_all = jnp.stack(wg_l)
    bg_all = jnp.stack(bg_l).reshape(2 * TENSORS_PER_HALF, 1, D)
    (w1, b1), (w2, b2) = params['combine']
    w1r = w1.reshape(3, D, D)
    b1r = b1.reshape(1, D)
    w2r = w2.reshape(1, D)
    b2r = b2.reshape(1, 1)

    rows0, node_e = _get_sc_gather(True)(emb, idx_halves[0], nidx)
    rows1, = _get_sc_gather(False)(emb, idx_halves[1])
    node_e = node_e.reshape(B, D)
    m = []
    for l, rows in enumerate((rows0, rows1)):
        m.append(_message_pass_means(
            rows, ws_h[l], wm_h[l], b_h[l], w_halves[l]
        ).reshape(TENSORS_PER_HALF, B, D))
    return _combine(m[0], m[1], node_e, wg_all, bg_all, w1r, b1r, w2r, b2r)


# speedup vs baseline: 3.5011x; 1.0007x over previous
import functools

import jax
import jax.numpy as jnp
from jax import lax
from jax.experimental import pallas as pl
from jax.experimental.pallas import tpu as pltpu
from jax.experimental.pallas import tpu_sc as plsc

D = 128
B = 256
T = 32
NLAYERS = 2
ROWS_PER_TENSOR = B * T * 3
TENSORS_PER_HALF = 3
HROWS = TENSORS_PER_HALF * ROWS_PER_TENSOR

NW = 32
CHUNK = 128
H_CHUNKS = HROWS // (NW * CHUNK)
NODE_PER_WORKER = B // NW

RB = 3072
BATCHES_PER_BLOCK = RB // (T * 3)
H_BLOCKS = HROWS // RB


def _sc_gather_half_body(with_node, table, idx, nidx, out, nout, idx_v, nidx_v,
                         rows_a, rows_b, nrows_v, sem_a, sem_b):
    wid = lax.axis_index("s") * 2 + lax.axis_index("c")
    pltpu.sync_copy(idx.at[wid], idx_v)

    def store(c, rows):
        ch = wid * H_CHUNKS + c
        pltpu.sync_copy(rows, out.at[ch // 192, (ch // 64) % 3, ch % 64])

    pltpu.async_copy(table.at[idx_v.at[0]], rows_a, sem_a)

    def pair(i, _):
        c = 2 * i
        pltpu.async_copy(table.at[idx_v.at[c + 1]], rows_b, sem_b)
        pltpu.make_async_copy(table.at[idx_v.at[c]], rows_a, sem_a).wait()
        store(c, rows_a)

        @pl.when(c + 2 < H_CHUNKS)
        def _():
            pltpu.async_copy(table.at[idx_v.at[c + 2]], rows_a, sem_a)

        pltpu.make_async_copy(table.at[idx_v.at[c + 1]], rows_b, sem_b).wait()
        store(c + 1, rows_b)
        return 0

    lax.fori_loop(0, H_CHUNKS // 2, pair, 0)

    if with_node:
        pltpu.sync_copy(nidx.at[wid], nidx_v)
        pltpu.async_copy(table.at[nidx_v], nrows_v, sem_a).wait()
        pltpu.sync_copy(nrows_v, nout.at[wid])


@functools.cache
def _get_sc_gather(with_node):
    out_type = [jax.ShapeDtypeStruct(
        (TENSORS_PER_HALF, 3, HROWS // (9 * CHUNK), CHUNK, D), jnp.float32)]
    scratch = [
        pltpu.VMEM((H_CHUNKS, CHUNK), jnp.int32),
        pltpu.VMEM((NODE_PER_WORKER,), jnp.int32),
        pltpu.VMEM((CHUNK, D), jnp.float32),
        pltpu.VMEM((CHUNK, D), jnp.float32),
        pltpu.VMEM((NODE_PER_WORKER, D), jnp.float32),
        pltpu.SemaphoreType.DMA,
        pltpu.SemaphoreType.DMA,
    ]
    if with_node:
        out_type.append(jax.ShapeDtypeStruct((NW, NODE_PER_WORKER, D), jnp.float32))

        def body(table, idx, nidx, out, nout, idx_v, nidx_v, ra, rb, nv, sa, sb):
            _sc_gather_half_body(True, table, idx, nidx, out, nout, idx_v,
                                 nidx_v, ra, rb, nv, sa, sb)
    else:
        def body(table, idx, out, idx_v, nidx_v, ra, rb, nv, sa, sb):
            _sc_gather_half_body(False, table, idx, None, out, None, idx_v,
                                 nidx_v, ra, rb, nv, sa, sb)

    return pl.kernel(
        body,
        out_type=tuple(out_type),
        mesh=plsc.VectorSubcoreMesh(core_axis_name="c", subcore_axis_name="s"),
        scratch_types=scratch,
        compiler_params=pltpu.CompilerParams(use_tc_tiling_on_sc=True),
    )


def _mp_body(x_ref, ws_ref, wm_ref, b_ref, w_ref, m_ref):
    g = RB // 3
    x3 = x_ref[0].reshape(3, g, D)
    ws = ws_ref[0]
    wm = wm_ref[0]
    b = b_ref[0]
    w3 = w_ref[0]
    s = x3[0] + x3[1] + x3[2]
    t = lax.dot(s, wm, preferred_element_type=jnp.float32) + b
    xf = x3.reshape(RB, D)
    tb = jnp.broadcast_to(t[None], (3, g, D)).reshape(RB, D)
    z = lax.dot(xf, ws - wm, preferred_element_type=jnp.float32) + tb
    y = (jnp.maximum(z, 0.0) * w3.reshape(RB, 1)).reshape(3, g, D)
    part = y[0] + y[1] + y[2]
    num = part.reshape(BATCHES_PER_BLOCK, T, D).sum(axis=1)
    wsum = w3[0] + w3[1] + w3[2]
    den = wsum.reshape(BATCHES_PER_BLOCK, T, 1).sum(axis=1)
    m_ref[...] = (num / (den + 1e-6))[None]


def _message_pass_means(rows, ws_all, wm_all, b_all, w_all):
    blks = RB // 3 // CHUNK
    return pl.pallas_call(
        _mp_body,
        grid=(H_BLOCKS,),
        in_specs=[
            pl.BlockSpec((1, 3, blks, CHUNK, D), lambda i: (i // 8, 0, i % 8, 0, 0)),
            pl.BlockSpec((1, D, D), lambda i: (i // 8, 0, 0)),
            pl.BlockSpec((1, D, D), lambda i: (i // 8, 0, 0)),
            pl.BlockSpec((1, 1, D), lambda i: (i // 8, 0, 0)),
            pl.BlockSpec((1, 3, RB // 3, 1), lambda i: (i // 8, 0, i % 8, 0)),
        ],
        out_specs=pl.BlockSpec((1, BATCHES_PER_BLOCK, D), lambda i: (i, 0, 0)),
        out_shape=jax.ShapeDtypeStruct((H_BLOCKS, BATCHES_PER_BLOCK, D), jnp.float32),
    )(rows, ws_all, wm_all, b_all, w_all)


def _combine_body(ma_ref, mb_ref, ne_ref, wg_ref, bg_ref, w1_ref, b1_ref,
                  w2_ref, b2_ref, o_ref):
    def agg(m_ref, half, k):
        return (lax.dot(m_ref[k], wg_ref[half * 3 + k],
                        preferred_element_type=jnp.float32)
                + bg_ref[half * 3 + k])

    t_mean = 0.25 * (agg(ma_ref, 0, 0) - agg(ma_ref, 0, 1)
                     + agg(mb_ref, 1, 0) - agg(mb_ref, 1, 1))
    n_mean = 0.5 * (agg(ma_ref, 0, 2) + agg(mb_ref, 1, 2))
    h = (lax.dot(ne_ref[...], w1_ref[0], preferred_element_type=jnp.float32)
         + lax.dot(t_mean, w1_ref[1], preferred_element_type=jnp.float32)
         + lax.dot(n_mean, w1_ref[2], preferred_element_type=jnp.float32)
         + b1_ref[...])
    h1 = jnp.maximum(h, 0.0)
    o_ref[...] = jnp.sum(h1 * w2_ref[...], axis=1, keepdims=True) + b2_ref[...]


def _combine(ma, mb, node_e, wg_all, bg_all, w1, b1, w2, b2):
    return pl.pallas_call(
        _combine_body,
        out_shape=jax.ShapeDtypeStruct((B, 1), jnp.float32),
    )(ma, mb, node_e, wg_all, bg_all, w1, b1, w2, b2)


def kernel(params, node_id, trianglelogic, squarelogic, triangle, notriangle,
           square, nosquare, triangle_neighbor, triangle_mask, square_neighbor,
           square_mask):
    emb = params['embedding']

    idx_halves, w_halves, ws_h, wm_h, b_h = [], [], [], [], []
    wg_l, bg_l = [], []
    ones = jnp.ones((ROWS_PER_TENSOR,), jnp.float32)

    def node_major(a):
        return jnp.transpose(a, (2, 0, 1)).reshape(-1)

    for l in range(NLAYERS):
        idx_halves.append(jnp.concatenate([
            node_major(triangle[:, l]),
            node_major(notriangle[:, l]),
            node_major(triangle_neighbor[:, l]),
        ]).astype(jnp.int32).reshape(NW, H_CHUNKS, CHUNK))
        w_halves.append(jnp.concatenate(
            [ones, ones, node_major(triangle_mask[:, l])]
        ).reshape(TENSORS_PER_HALF, 3, B * T, 1))
        (ws_t, wm_t, bias_t) = params['tmp'][l][0]
        (ws_n, wm_n, bias_n) = params['tnp'][l][0]
        ws_h.append(jnp.stack([ws_t, ws_t, ws_n]))
        wm_h.append(jnp.stack([wm_t, wm_t, wm_n]))
        b_h.append(jnp.stack([bias_t, bias_t, bias_n]).reshape(3, 1, D))
        for name in ('tpa', 'tga', 'tna'):
            wg, bg = params[name][l][0]
            wg_l.append(wg)
            bg_l.append(bg)

    nidx = node_id.astype(jnp.int32).reshape(NW, NODE_PER_WORKER)
    wg_all = jnp.stack(wg_l)
    bg_all = jnp.stack(bg_l).reshape(2 * TENSORS_PER_HALF, 1, D)
    (w1, b1), (w2, b2) = params['combine']
    w1r = w1.reshape(3, D, D)
    b1r = b1.reshape(1, D)
    w2r = w2.reshape(1, D)
    b2r = b2.reshape(1, 1)

    rows0, node_e = _get_sc_gather(True)(emb, idx_halves[0], nidx)
    rows1, = _get_sc_gather(False)(emb, idx_halves[1])
    node_e = node_e.reshape(B, D)
    m = []
    for l, rows in enumerate((rows0, rows1)):
        m.append(_message_pass_means(
            rows, ws_h[l], wm_h[l], b_h[l], w_halves[l]
        ).reshape(TENSORS_PER_HALF, B, D))
    return _combine(m[0], m[1], node_e, wg_all, bg_all, w1r, b1r, w2r, b2r)


# speedup vs baseline: 4.9529x; 1.4146x over previous
import functools

import jax
import jax.numpy as jnp
from jax import lax
from jax.experimental import pallas as pl
from jax.experimental.pallas import tpu as pltpu
from jax.experimental.pallas import tpu_sc as plsc

D = 128
B = 256
T = 32
NLAYERS = 2
ROWS_PER_TENSOR = B * T * 3
TENSORS_PER_HALF = 3
HROWS = TENSORS_PER_HALF * ROWS_PER_TENSOR

NW = 32
CHUNK = 128
H_CHUNKS = HROWS // (NW * CHUNK)
NODE_PER_WORKER = B // NW

RB = 3072
BATCHES_PER_BLOCK = RB // (T * 3)
H_BLOCKS = HROWS // RB


def _sc_gather_half_body(with_node, table, idx, nidx, out, nout, idx_v, nidx_v,
                         rows_a, rows_b, nrows_v, sem_a, sem_b):
    wid = lax.axis_index("s") * 2 + lax.axis_index("c")
    pltpu.sync_copy(idx.at[wid], idx_v)

    def store(c, rows):
        ch = wid * H_CHUNKS + c
        pltpu.sync_copy(rows, out.at[ch // 192, (ch // 64) % 3, ch % 64])

    pltpu.async_copy(table.at[idx_v.at[0]], rows_a, sem_a)

    def pair(i, _):
        c = 2 * i
        pltpu.async_copy(table.at[idx_v.at[c + 1]], rows_b, sem_b)
        pltpu.make_async_copy(table.at[idx_v.at[c]], rows_a, sem_a).wait()
        store(c, rows_a)

        @pl.when(c + 2 < H_CHUNKS)
        def _():
            pltpu.async_copy(table.at[idx_v.at[c + 2]], rows_a, sem_a)

        pltpu.make_async_copy(table.at[idx_v.at[c + 1]], rows_b, sem_b).wait()
        store(c + 1, rows_b)
        return 0

    lax.fori_loop(0, H_CHUNKS // 2, pair, 0)

    if with_node:
        pltpu.sync_copy(nidx.at[wid], nidx_v)
        pltpu.async_copy(table.at[nidx_v], nrows_v, sem_a).wait()
        pltpu.sync_copy(nrows_v, nout.at[wid])


@functools.cache
def _get_sc_gather(with_node):
    out_type = [jax.ShapeDtypeStruct(
        (TENSORS_PER_HALF, 3, HROWS // (9 * CHUNK), CHUNK, D), jnp.float32)]
    scratch = [
        pltpu.VMEM((H_CHUNKS, CHUNK), jnp.int32),
        pltpu.VMEM((NODE_PER_WORKER,), jnp.int32),
        pltpu.VMEM((CHUNK, D), jnp.float32),
        pltpu.VMEM((CHUNK, D), jnp.float32),
        pltpu.VMEM((NODE_PER_WORKER, D), jnp.float32),
        pltpu.SemaphoreType.DMA,
        pltpu.SemaphoreType.DMA,
    ]
    if with_node:
        out_type.append(jax.ShapeDtypeStruct((NW, NODE_PER_WORKER, D), jnp.float32))

        def body(table, idx, nidx, out, nout, idx_v, nidx_v, ra, rb, nv, sa, sb):
            _sc_gather_half_body(True, table, idx, nidx, out, nout, idx_v,
                                 nidx_v, ra, rb, nv, sa, sb)
    else:
        def body(table, idx, out, idx_v, nidx_v, ra, rb, nv, sa, sb):
            _sc_gather_half_body(False, table, idx, None, out, None, idx_v,
                                 nidx_v, ra, rb, nv, sa, sb)

    return pl.kernel(
        body,
        out_type=tuple(out_type),
        mesh=plsc.VectorSubcoreMesh(core_axis_name="c", subcore_axis_name="s"),
        scratch_types=scratch,
        compiler_params=pltpu.CompilerParams(use_tc_tiling_on_sc=True),
    )


def _mp_body(x_ref, ws_ref, wm_ref, b_ref, wmat_ref, m_ref):
    g = RB // 3
    x3 = x_ref[0].reshape(3, g, D)
    ws = ws_ref[0]
    wm = wm_ref[0]
    b = b_ref[0]
    wmat = wmat_ref[0, 0]
    s = x3[0] + x3[1] + x3[2]
    t = lax.dot(s, wm, preferred_element_type=jnp.float32) + b
    xf = x3.reshape(RB, D)
    tb = jnp.broadcast_to(t[None], (3, g, D)).reshape(RB, D)
    z = lax.dot(xf, ws - wm, preferred_element_type=jnp.float32) + tb
    y = jnp.maximum(z, 0.0)
    num = lax.dot(wmat, y, preferred_element_type=jnp.float32)
    den = jnp.sum(wmat, axis=1, keepdims=True)
    m_ref[...] = (num / (den + 1e-6))[None]


def _message_pass_means(rows, ws_all, wm_all, b_all, wmat):
    blks = RB // 3 // CHUNK
    return pl.pallas_call(
        _mp_body,
        grid=(H_BLOCKS,),
        in_specs=[
            pl.BlockSpec((1, 3, blks, CHUNK, D), lambda i: (i // 8, 0, i % 8, 0, 0)),
            pl.BlockSpec((1, D, D), lambda i: (i // 8, 0, 0)),
            pl.BlockSpec((1, D, D), lambda i: (i // 8, 0, 0)),
            pl.BlockSpec((1, 1, D), lambda i: (i // 8, 0, 0)),
            pl.BlockSpec((1, 1, BATCHES_PER_BLOCK, RB),
                         lambda i: (i // 8, i % 8, 0, 0)),
        ],
        out_specs=pl.BlockSpec((1, BATCHES_PER_BLOCK, D), lambda i: (i, 0, 0)),
        out_shape=jax.ShapeDtypeStruct((H_BLOCKS, BATCHES_PER_BLOCK, D), jnp.float32),
    )(rows, ws_all, wm_all, b_all, wmat)


def _combine_body(ma_ref, mb_ref, ne_ref, wg_ref, bg_ref, w1_ref, b1_ref,
                  w2_ref, b2_ref, o_ref):
    def agg(m_ref, half, k):
        return (lax.dot(m_ref[k], wg_ref[half * 3 + k],
                        preferred_element_type=jnp.float32)
                + bg_ref[half * 3 + k])

    t_mean = 0.25 * (agg(ma_ref, 0, 0) - agg(ma_ref, 0, 1)
                     + agg(mb_ref, 1, 0) - agg(mb_ref, 1, 1))
    n_mean = 0.5 * (agg(ma_ref, 0, 2) + agg(mb_ref, 1, 2))
    h = (lax.dot(ne_ref[...], w1_ref[0], preferred_element_type=jnp.float32)
         + lax.dot(t_mean, w1_ref[1], preferred_element_type=jnp.float32)
         + lax.dot(n_mean, w1_ref[2], preferred_element_type=jnp.float32)
         + b1_ref[...])
    h1 = jnp.maximum(h, 0.0)
    o_ref[...] = jnp.sum(h1 * w2_ref[...], axis=1, keepdims=True) + b2_ref[...]


def _combine(ma, mb, node_e, wg_all, bg_all, w1, b1, w2, b2):
    return pl.pallas_call(
        _combine_body,
        out_shape=jax.ShapeDtypeStruct((B, 1), jnp.float32),
    )(ma, mb, node_e, wg_all, bg_all, w1, b1, w2, b2)


def kernel(params, node_id, trianglelogic, squarelogic, triangle, notriangle,
           square, nosquare, triangle_neighbor, triangle_mask, square_neighbor,
           square_mask):
    emb = params['embedding']

    idx_halves, w_halves, ws_h, wm_h, b_h = [], [], [], [], []
    wg_l, bg_l = [], []
    eye = jnp.eye(BATCHES_PER_BLOCK, dtype=jnp.float32)

    def node_major(a):
        return jnp.transpose(a, (2, 0, 1)).reshape(-1)

    def build_wmat(mask_l):
        mt = jnp.transpose(mask_l, (2, 0, 1)).reshape(3, 8, BATCHES_PER_BLOCK, T)
        w_base = jnp.concatenate([
            jnp.ones((2, 3, 8, BATCHES_PER_BLOCK, T), jnp.float32), mt[None]],
            axis=0)
        wm6 = (w_base.transpose(0, 2, 1, 3, 4)[:, :, None]
               * eye[None, None, :, None, :, None])
        return wm6.reshape(TENSORS_PER_HALF, 8, BATCHES_PER_BLOCK, RB)

    for l in range(NLAYERS):
        idx_halves.append(jnp.concatenate([
            node_major(triangle[:, l]),
            node_major(notriangle[:, l]),
            node_major(triangle_neighbor[:, l]),
        ]).astype(jnp.int32).reshape(NW, H_CHUNKS, CHUNK))
        w_halves.append(build_wmat(triangle_mask[:, l]))
        (ws_t, wm_t, bias_t) = params['tmp'][l][0]
        (ws_n, wm_n, bias_n) = params['tnp'][l][0]
        ws_h.append(jnp.stack([ws_t, ws_t, ws_n]))
        wm_h.append(jnp.stack([wm_t, wm_t, wm_n]))
        b_h.append(jnp.stack([bias_t, bias_t, bias_n]).reshape(3, 1, D))
        for name in ('tpa', 'tga', 'tna'):
            wg, bg = params[name][l][0]
            wg_l.append(wg)
            bg_l.append(bg)

    nidx = node_id.astype(jnp.int32).reshape(NW, NODE_PER_WORKER)
    wg_all = jnp.stack(wg_l)
    bg_all = jnp.stack(bg_l).reshape(2 * TENSORS_PER_HALF, 1, D)
    (w1, b1), (w2, b2) = params['combine']
    w1r = w1.reshape(3, D, D)
    b1r = b1.reshape(1, D)
    w2r = w2.reshape(1, D)
    b2r = b2.reshape(1, 1)

    rows0, node_e = _get_sc_gather(True)(emb, idx_halves[0], nidx)
    rows1, = _get_sc_gather(False)(emb, idx_halves[1])
    node_e = node_e.reshape(B, D)
    m = []
    for l, rows in enumerate((rows0, rows1)):
        m.append(_message_pass_means(
            rows, ws_h[l], wm_h[l], b_h[l], w_halves[l]
        ).reshape(TENSORS_PER_HALF, B, D))
    return _combine(m[0], m[1], node_e, wg_all, bg_all, w1r, b1r, w2r, b2r)


# speedup vs baseline: 5.1848x; 1.0468x over previous
import functools

import jax
import jax.numpy as jnp
from jax import lax
from jax.experimental import pallas as pl
from jax.experimental.pallas import tpu as pltpu
from jax.experimental.pallas import tpu_sc as plsc

D = 128
B = 256
T = 32
NLAYERS = 2
ROWS_PER_TENSOR = B * T * 3
TENSORS_PER_HALF = 3
HROWS = TENSORS_PER_HALF * ROWS_PER_TENSOR

NW = 32
CHUNK = 128
H_CHUNKS = HROWS // (NW * CHUNK)
NODE_PER_WORKER = B // NW

RB = 3072
BATCHES_PER_BLOCK = RB // (T * 3)
H_BLOCKS = HROWS // RB


def _sc_gather_half_body(with_node, table, idx, nidx, out, nout, idx_v, nidx_v,
                         rows_a, rows_b, nrows_v, sem_a, sem_b):
    wid = lax.axis_index("s") * 2 + lax.axis_index("c")
    pltpu.sync_copy(idx.at[wid], idx_v)

    def store(c, rows):
        ch = wid * H_CHUNKS + c
        pltpu.sync_copy(rows, out.at[ch // 192, (ch // 64) % 3, ch % 64])

    pltpu.async_copy(table.at[idx_v.at[0]], rows_a, sem_a)

    def pair(i, _):
        c = 2 * i
        pltpu.async_copy(table.at[idx_v.at[c + 1]], rows_b, sem_b)
        pltpu.make_async_copy(table.at[idx_v.at[c]], rows_a, sem_a).wait()
        store(c, rows_a)

        @pl.when(c + 2 < H_CHUNKS)
        def _():
            pltpu.async_copy(table.at[idx_v.at[c + 2]], rows_a, sem_a)

        pltpu.make_async_copy(table.at[idx_v.at[c + 1]], rows_b, sem_b).wait()
        store(c + 1, rows_b)
        return 0

    lax.fori_loop(0, H_CHUNKS // 2, pair, 0)

    if with_node:
        pltpu.sync_copy(nidx.at[wid], nidx_v)
        pltpu.async_copy(table.at[nidx_v], nrows_v, sem_a).wait()
        pltpu.sync_copy(nrows_v, nout.at[wid])


@functools.cache
def _get_sc_gather(with_node):
    out_type = [jax.ShapeDtypeStruct(
        (TENSORS_PER_HALF, 3, HROWS // (9 * CHUNK), CHUNK, D), jnp.float32)]
    scratch = [
        pltpu.VMEM((H_CHUNKS, CHUNK), jnp.int32),
        pltpu.VMEM((NODE_PER_WORKER,), jnp.int32),
        pltpu.VMEM((CHUNK, D), jnp.float32),
        pltpu.VMEM((CHUNK, D), jnp.float32),
        pltpu.VMEM((NODE_PER_WORKER, D), jnp.float32),
        pltpu.SemaphoreType.DMA,
        pltpu.SemaphoreType.DMA,
    ]
    if with_node:
        out_type.append(jax.ShapeDtypeStruct((NW, NODE_PER_WORKER, D), jnp.float32))

        def body(table, idx, nidx, out, nout, idx_v, nidx_v, ra, rb, nv, sa, sb):
            _sc_gather_half_body(True, table, idx, nidx, out, nout, idx_v,
                                 nidx_v, ra, rb, nv, sa, sb)
    else:
        def body(table, idx, out, idx_v, nidx_v, ra, rb, nv, sa, sb):
            _sc_gather_half_body(False, table, idx, None, out, None, idx_v,
                                 nidx_v, ra, rb, nv, sa, sb)

    return pl.kernel(
        body,
        out_type=tuple(out_type),
        mesh=plsc.VectorSubcoreMesh(core_axis_name="c", subcore_axis_name="s"),
        scratch_types=scratch,
        compiler_params=pltpu.CompilerParams(use_tc_tiling_on_sc=True),
    )


def _graphlet_relu(x_ref, ws_ref, wm_ref, b_ref):
    g = RB // 3
    x3 = x_ref[0].reshape(3, g, D)
    ws = ws_ref[0]
    wm = wm_ref[0]
    b = b_ref[0]
    s = x3[0] + x3[1] + x3[2]
    t = lax.dot(s, wm, preferred_element_type=jnp.float32) + b
    xf = x3.reshape(RB, D)
    tb = jnp.broadcast_to(t[None], (3, g, D)).reshape(RB, D)
    z = lax.dot(xf, ws - wm, preferred_element_type=jnp.float32) + tb
    return jnp.maximum(z, 0.0)


def _mp_plain_body(x_ref, ws_ref, wm_ref, b_ref, m_ref):
    g = RB // 3
    y = _graphlet_relu(x_ref, ws_ref, wm_ref, b_ref).reshape(3, g, D)
    part = y[0] + y[1] + y[2]
    num = part.reshape(BATCHES_PER_BLOCK, T, D).sum(axis=1)
    m_ref[...] = (num * (1.0 / (T * 3 + 1e-6)))[None]


def _mp_masked_body(x_ref, ws_ref, wm_ref, b_ref, wmat_ref, m_ref):
    y = _graphlet_relu(x_ref, ws_ref, wm_ref, b_ref)
    wmat = wmat_ref[0]
    num = lax.dot(wmat, y, preferred_element_type=jnp.float32)
    den = jnp.sum(wmat, axis=1, keepdims=True)
    m_ref[...] = (num / (den + 1e-6))[None]


_BLKS = RB // 3 // CHUNK


def _mp_plain(rows, ws_all, wm_all, b_all):
    return pl.pallas_call(
        _mp_plain_body,
        grid=(16,),
        in_specs=[
            pl.BlockSpec((1, 3, _BLKS, CHUNK, D), lambda i: (i // 8, 0, i % 8, 0, 0)),
            pl.BlockSpec((1, D, D), lambda i: (i // 8, 0, 0)),
            pl.BlockSpec((1, D, D), lambda i: (i // 8, 0, 0)),
            pl.BlockSpec((1, 1, D), lambda i: (i // 8, 0, 0)),
        ],
        out_specs=pl.BlockSpec((1, BATCHES_PER_BLOCK, D), lambda i: (i, 0, 0)),
        out_shape=jax.ShapeDtypeStruct((16, BATCHES_PER_BLOCK, D), jnp.float32),
    )(rows, ws_all, wm_all, b_all)


def _mp_masked(rows, ws_all, wm_all, b_all, wmat):
    return pl.pallas_call(
        _mp_masked_body,
        grid=(8,),
        in_specs=[
            pl.BlockSpec((1, 3, _BLKS, CHUNK, D), lambda i: (2, 0, i, 0, 0)),
            pl.BlockSpec((1, D, D), lambda i: (2, 0, 0)),
            pl.BlockSpec((1, D, D), lambda i: (2, 0, 0)),
            pl.BlockSpec((1, 1, D), lambda i: (2, 0, 0)),
            pl.BlockSpec((1, BATCHES_PER_BLOCK, RB), lambda i: (i, 0, 0)),
        ],
        out_specs=pl.BlockSpec((1, BATCHES_PER_BLOCK, D), lambda i: (i, 0, 0)),
        out_shape=jax.ShapeDtypeStruct((8, BATCHES_PER_BLOCK, D), jnp.float32),
    )(rows, ws_all, wm_all, b_all, wmat)


def _combine_body(pa_ref, na_ref, pb_ref, nb_ref, ne_ref, wg_ref, bg_ref,
                  w1_ref, b1_ref, w2_ref, b2_ref, o_ref):
    def agg(m, k):
        return lax.dot(m, wg_ref[k], preferred_element_type=jnp.float32) + bg_ref[k]

    t_mean = 0.25 * (agg(pa_ref[0], 0) - agg(pa_ref[1], 1)
                     + agg(pb_ref[0], 3) - agg(pb_ref[1], 4))
    n_mean = 0.5 * (agg(na_ref[...], 2) + agg(nb_ref[...], 5))
    h = (lax.dot(ne_ref[...], w1_ref[0], preferred_element_type=jnp.float32)
         + lax.dot(t_mean, w1_ref[1], preferred_element_type=jnp.float32)
         + lax.dot(n_mean, w1_ref[2], preferred_element_type=jnp.float32)
         + b1_ref[...])
    h1 = jnp.maximum(h, 0.0)
    o_ref[...] = jnp.sum(h1 * w2_ref[...], axis=1, keepdims=True) + b2_ref[...]


def _combine(pa, na, pb, nb, node_e, wg_all, bg_all, w1, b1, w2, b2):
    return pl.pallas_call(
        _combine_body,
        out_shape=jax.ShapeDtypeStruct((B, 1), jnp.float32),
    )(pa, na, pb, nb, node_e, wg_all, bg_all, w1, b1, w2, b2)


def kernel(params, node_id, trianglelogic, squarelogic, triangle, notriangle,
           square, nosquare, triangle_neighbor, triangle_mask, square_neighbor,
           square_mask):
    emb = params['embedding']

    idx_halves, w_halves, ws_h, wm_h, b_h = [], [], [], [], []
    wg_l, bg_l = [], []
    eye = jnp.eye(BATCHES_PER_BLOCK, dtype=jnp.float32)

    def node_major(a):
        return jnp.transpose(a, (2, 0, 1)).reshape(-1)

    def build_wmat(mask_l):
        mt = jnp.transpose(mask_l, (2, 0, 1)).reshape(3, 8, BATCHES_PER_BLOCK, T)
        wm5 = (mt.transpose(1, 0, 2, 3)[:, None]
               * eye[None, :, None, :, None])
        return wm5.reshape(8, BATCHES_PER_BLOCK, RB)

    for l in range(NLAYERS):
        idx_halves.append(jnp.concatenate([
            node_major(triangle[:, l]),
            node_major(notriangle[:, l]),
            node_major(triangle_neighbor[:, l]),
        ]).astype(jnp.int32).reshape(NW, H_CHUNKS, CHUNK))
        w_halves.append(build_wmat(triangle_mask[:, l]))
        (ws_t, wm_t, bias_t) = params['tmp'][l][0]
        (ws_n, wm_n, bias_n) = params['tnp'][l][0]
        ws_h.append(jnp.stack([ws_t, ws_t, ws_n]))
        wm_h.append(jnp.stack([wm_t, wm_t, wm_n]))
        b_h.append(jnp.stack([bias_t, bias_t, bias_n]).reshape(3, 1, D))
        for name in ('tpa', 'tga', 'tna'):
            wg, bg = params[name][l][0]
            wg_l.append(wg)
            bg_l.append(bg)

    nidx = node_id.astype(jnp.int32).reshape(NW, NODE_PER_WORKER)
    wg_all = jnp.stack(wg_l)
    bg_all = jnp.stack(bg_l).reshape(2 * TENSORS_PER_HALF, 1, D)
    (w1, b1), (w2, b2) = params['combine']
    w1r = w1.reshape(3, D, D)
    b1r = b1.reshape(1, D)
    w2r = w2.reshape(1, D)
    b2r = b2.reshape(1, 1)

    rows0, node_e = _get_sc_gather(True)(emb, idx_halves[0], nidx)
    rows1, = _get_sc_gather(False)(emb, idx_halves[1])
    node_e = node_e.reshape(B, D)
    mp, mn = [], []
    for l, rows in enumerate((rows0, rows1)):
        mp.append(_mp_plain(rows, ws_h[l], wm_h[l], b_h[l]).reshape(2, B, D))
        mn.append(_mp_masked(rows, ws_h[l], wm_h[l], b_h[l],
                             w_halves[l]).reshape(B, D))
    return _combine(mp[0], mn[0], mp[1], mn[1], node_e, wg_all, bg_all,
                    w1r, b1r, w2r, b2r)
